# Initial kernel scaffold; baseline (speedup 1.0000x reference)
import jax
import jax.numpy as jnp
from jax.experimental import pallas as pl


def kernel(feature_map, ground_truth):
    raise NotImplementedError("write your pallas kernel here")


# speedup vs baseline: 1.0318x; 1.0318x over previous
import jax
import jax.numpy as jnp
from jax.experimental import pallas as pl


def _sums_body(f_ref, g_ref, sc_ref, st_ref, cc_ref):
    b = pl.program_id(0)
    h = pl.program_id(1)
    x = f_ref[0]
    m = (g_ref[0, 0] == 1).astype(jnp.float32)
    sc = jnp.sum(x * m[None, :], axis=1)
    st = jnp.sum(x, axis=1)
    cc = jnp.sum(m)

    @pl.when((b == 0) & (h == 0))
    def _init():
        sc_ref[...] = jnp.zeros_like(sc_ref)
        st_ref[...] = jnp.zeros_like(st_ref)
        cc_ref[...] = jnp.zeros_like(cc_ref)

    sc_ref[b] += sc
    st_ref[b] += st
    cc_ref[b] += cc


def _out_body(f_ref, g_ref, d_ref, bias_ref, o_ref):
    b = pl.program_id(0)
    x = f_ref[0]
    g = g_ref[0, 0]
    dv = d_ref[b]
    dot = jnp.sum(x * dv[:, None], axis=0)
    ss = jnp.sum(x * x, axis=0)
    nrm = jnp.maximum(jnp.sqrt(ss), 1e-12)
    z = dot / nrm
    z = jnp.where(g == 1, z, -z)
    o_ref[0, 0] = jnp.exp(z + bias_ref[b, 0])


def kernel(feature_map, ground_truth):
    B, C, H, W = feature_map.shape
    HW = H * W
    S = 8192
    nS = HW // S

    f2 = feature_map.reshape(B, C, HW)
    g2 = ground_truth.reshape(B, 1, HW)

    sum_c, sum_t, cnt_v = pl.pallas_call(
        _sums_body,
        grid=(B, nS),
        in_specs=[
            pl.BlockSpec((1, C, S), lambda b, h: (b, 0, h)),
            pl.BlockSpec((1, 1, S), lambda b, h: (b, 0, h)),
        ],
        out_specs=[
            pl.BlockSpec((B, C), lambda b, h: (0, 0)),
            pl.BlockSpec((B, C), lambda b, h: (0, 0)),
            pl.BlockSpec((B, C), lambda b, h: (0, 0)),
        ],
        out_shape=[
            jax.ShapeDtypeStruct((B, C), jnp.float32),
            jax.ShapeDtypeStruct((B, C), jnp.float32),
            jax.ShapeDtypeStruct((B, C), jnp.float32),
        ],
    )(f2, g2)

    cnt_c = cnt_v[:, 0]
    cnt_n = HW - cnt_c
    sum_n = sum_t - sum_c
    valid = (cnt_c > 0) & (cnt_n > 0)
    pc = sum_c / jnp.maximum(cnt_c, 1.0)[:, None]
    pn = sum_n / jnp.maximum(cnt_n, 1.0)[:, None]
    pcn = pc / jnp.maximum(jnp.linalg.norm(pc, axis=1, keepdims=True), 1e-12)
    pnn = pn / jnp.maximum(jnp.linalg.norm(pn, axis=1, keepdims=True), 1e-12)
    d = pcn - pnn
    bias = jnp.where(valid, 0.0, -jnp.inf).astype(jnp.float32)
    bias_v = jnp.broadcast_to(bias[:, None], (B, C))

    out = pl.pallas_call(
        _out_body,
        grid=(B, nS),
        in_specs=[
            pl.BlockSpec((1, C, S), lambda b, h: (b, 0, h)),
            pl.BlockSpec((1, 1, S), lambda b, h: (b, 0, h)),
            pl.BlockSpec((B, C), lambda b, h: (0, 0)),
            pl.BlockSpec((B, C), lambda b, h: (0, 0)),
        ],
        out_specs=pl.BlockSpec((1, 1, S), lambda b, h: (b, 0, h)),
        out_shape=jax.ShapeDtypeStruct((B, 1, HW), jnp.float32),
    )(f2, g2, d, bias_v)

    return out.reshape(B, H, W)
